# Initial kernel scaffold; baseline (speedup 1.0000x reference)
import jax
import jax.numpy as jnp
from jax.experimental import pallas as pl


def kernel(x, proxies, Wq, bq, Wk, bk, Wv, bv, Wo, bo, Wfc, bfc):
    raise NotImplementedError("write your pallas kernel here")


# speedup vs baseline: 2777.5576x; 2777.5576x over previous
import jax
import jax.numpy as jnp
from jax.experimental import pallas as pl

_P = 64
_S = 4096
_D = 128
_H = 64
_ODIM = 64
_SCALE = 1.0 / (_H ** 0.5)


def _gnn_kernel(x_ref, p_ref, wq_ref, bq_ref, wk_ref, bk_ref, wv_ref, bv_ref,
                wo_ref, bo_ref, wfc_ref, bfc_ref, preds_ref, feats_ref):
    xb = x_ref[...]
    q = jnp.dot(xb, wq_ref[...], preferred_element_type=jnp.float32) + bq_ref[...]
    pr = p_ref[...]
    k = jnp.dot(pr, wk_ref[...], preferred_element_type=jnp.float32) + bk_ref[...]
    v = jnp.dot(pr, wv_ref[...], preferred_element_type=jnp.float32) + bv_ref[...]
    agg_parts = []
    for hd in range(2):
        sl = slice(hd * _H, (hd + 1) * _H)
        s = jax.lax.dot_general(q[:, sl], k[:, sl], (((1,), (1,)), ((), ())),
                                preferred_element_type=jnp.float32) * _SCALE
        m = jnp.max(s, axis=1, keepdims=True)
        e = jnp.exp(s - m)
        a = e / jnp.sum(e, axis=1, keepdims=True)
        agg_parts.append(jnp.dot(a, v[:, sl], preferred_element_type=jnp.float32))
    agg = jnp.concatenate(agg_parts, axis=1)
    feats = jnp.maximum(
        jnp.dot(agg, wo_ref[...], preferred_element_type=jnp.float32) + bo_ref[...], 0.0)
    feats_ref[...] = feats
    preds_ref[...] = (
        jnp.dot(feats, wfc_ref[...], preferred_element_type=jnp.float32) + bfc_ref[...])


def kernel(x, proxies, Wq, bq, Wk, bk, Wv, bv, Wo, bo, Wfc, bfc):
    args = (x, proxies,
            Wq.T, bq.reshape(1, _D), Wk.T, bk.reshape(1, _D),
            Wv.T, bv.reshape(1, _D), Wo.T, bo.reshape(1, _D),
            Wfc.T, bfc.reshape(1, _ODIM))
    preds, feats = pl.pallas_call(
        _gnn_kernel,
        out_shape=(jax.ShapeDtypeStruct((_S, _ODIM), jnp.float32),
                   jax.ShapeDtypeStruct((_S, _D), jnp.float32)),
    )(*args)
    return preds, feats


# speedup vs baseline: 4566.4589x; 1.6441x over previous
import jax
import jax.numpy as jnp
from jax.experimental import pallas as pl

_P = 64
_S = 4096
_D = 128
_H = 64
_ODIM = 64
_SCALE = 1.0 / (_H ** 0.5)


def _dot_t(a, w):
    return jax.lax.dot_general(a, w, (((1,), (1,)), ((), ())),
                               preferred_element_type=jnp.float32)


def _gnn_kernel(x_ref, p_ref, wq_ref, bq_ref, wk_ref, bk_ref, wv_ref, bv_ref,
                wo_ref, bo_ref, wfc_ref, bfc_ref, preds_ref, feats_ref):
    xb = x_ref[...]
    q = _dot_t(xb, wq_ref[...]) + bq_ref[...]
    pr = p_ref[...]
    k = _dot_t(pr, wk_ref[...]) + bk_ref[...]
    v = _dot_t(pr, wv_ref[...]) + bv_ref[...]
    agg_parts = []
    for hd in range(2):
        sl = slice(hd * _H, (hd + 1) * _H)
        s = jax.lax.dot_general(q[:, sl], k[:, sl], (((1,), (1,)), ((), ())),
                                preferred_element_type=jnp.float32) * _SCALE
        m = jnp.max(s, axis=1, keepdims=True)
        e = jnp.exp(s - m)
        a = e / jnp.sum(e, axis=1, keepdims=True)
        agg_parts.append(jnp.dot(a, v[:, sl], preferred_element_type=jnp.float32))
    agg = jnp.concatenate(agg_parts, axis=1)
    feats = jnp.maximum(_dot_t(agg, wo_ref[...]) + bo_ref[...], 0.0)
    feats_ref[...] = feats
    preds_ref[...] = _dot_t(feats, wfc_ref[...]) + bfc_ref[...]


def kernel(x, proxies, Wq, bq, Wk, bk, Wv, bv, Wo, bo, Wfc, bfc):
    args = (x, proxies,
            Wq, bq.reshape(1, _D), Wk, bk.reshape(1, _D),
            Wv, bv.reshape(1, _D), Wo, bo.reshape(1, _D),
            Wfc, bfc.reshape(1, _ODIM))
    preds, feats = pl.pallas_call(
        _gnn_kernel,
        out_shape=(jax.ShapeDtypeStruct((_S, _ODIM), jnp.float32),
                   jax.ShapeDtypeStruct((_S, _D), jnp.float32)),
    )(*args)
    return preds, feats
